# baseline (speedup 1.0000x reference)
import functools

import jax
import jax.numpy as jnp
from jax import lax
from jax.experimental import pallas as pl
from jax.experimental.pallas import tpu as pltpu
from jax.experimental.pallas import tpu_sc as plsc

N_NODES = 10000
N_EDGES = 320000
D = 128
OUT = 3

NC = 2
NS = 16
NW = NC * NS
CH = 128
TOT_CH = 2560
C0 = 80
C1 = 80
TOTP = 2560
G_CHUNKS = TOTP // NW
EPAD = TOTP * CH
NPAD = 10240
RPT = NPAD // NS
RB = RPT // CH
NBLK = NPAD // 128

_f32 = jnp.float32


def _zero16():
    return jnp.zeros((16,), _f32)


def _one16():
    return jnp.ones((16,), _f32)


def _mesh():
    return plsc.VectorSubcoreMesh(
        core_axis_name="c", subcore_axis_name="s", num_cores=NC,
        num_subcores=NS,
    )


def _deg_body(dst_hbm, out_hbm, idx_v, ones_v, zeros_v, deg_sh):
    c = lax.axis_index("c")
    s = lax.axis_index("s")
    w = c * NS + s
    pltpu.sync_copy(dst_hbm.at[w], idx_v)
    for i in range(CH // 16):
        ones_v[pl.ds(i * 16, 16)] = _one16()
    for i in range(RPT // 16):
        zeros_v[pl.ds(i * 16, 16)] = _zero16()
    pltpu.sync_copy(zeros_v, deg_sh.at[pl.ds(s * RPT, RPT)])
    plsc.subcore_barrier()

    def chunk(g, carry):
        pltpu.sync_copy(ones_v, deg_sh.at[idx_v.at[g]], add=True)
        return carry

    lax.fori_loop(0, G_CHUNKS, chunk, 0)
    plsc.subcore_barrier()
    pltpu.sync_copy(deg_sh.at[pl.ds(s * RPT, RPT)],
                    out_hbm.at[c, pl.ds(s * RPT, RPT)])


@functools.cache
def _deg_kernel():
    return pl.kernel(
        _deg_body,
        out_type=jax.ShapeDtypeStruct((NC, NPAD), _f32),
        mesh=_mesh(),
        scratch_types=[
            pltpu.VMEM((G_CHUNKS, CH), jnp.int32),
            pltpu.VMEM((CH,), _f32),
            pltpu.VMEM((RPT,), _f32),
            pltpu.VMEM_SHARED((NPAD,), _f32),
        ],
    )


def _deg(dst_p):
    return _deg_kernel()(dst_p)


def _hop_body(ua_hbm, ub_hbm, src_hbm, dst_hbm, out_hbm, sidx_v, didx_v,
              rows_v, acc_sh):
    c = lax.axis_index("c")
    s = lax.axis_index("s")
    base = jnp.where(c == 0, s * C0, NS * C0 + s * C1)

    pltpu.sync_copy(src_hbm.at[pl.ds(base, C0)], sidx_v)
    pltpu.sync_copy(dst_hbm.at[pl.ds(base, C0)], didx_v)

    def zrow(i, carry):
        for k in range(D // 16):
            rows_v[i, pl.ds(k * 16, 16)] = _zero16()
        return carry

    lax.fori_loop(0, CH, zrow, 0)
    for r in range(RB):
        pltpu.sync_copy(rows_v, acc_sh.at[pl.ds(s * RPT + r * CH, CH)])
    plsc.subcore_barrier()

    def make_chunk(u_hbm):
        def chunk(g, carry):
            pltpu.sync_copy(u_hbm.at[sidx_v.at[g]], rows_v)
            pltpu.sync_copy(rows_v, acc_sh.at[didx_v.at[g]], add=True)
            return carry
        return chunk

    @pl.when(c == 0)
    def _():
        lax.fori_loop(0, C0, make_chunk(ua_hbm), 0)

    @pl.when(c == 1)
    def _():
        lax.fori_loop(0, C1, make_chunk(ub_hbm), 0)

    plsc.subcore_barrier()
    for r in range(RB):
        pltpu.sync_copy(acc_sh.at[pl.ds(s * RPT + r * CH, CH)],
                        out_hbm.at[c, pl.ds(s * RPT + r * CH, CH)])


@functools.cache
def _hop_kernel():
    return pl.kernel(
        _hop_body,
        out_type=jax.ShapeDtypeStruct((NC, NPAD, D), _f32),
        mesh=_mesh(),
        scratch_types=[
            pltpu.VMEM((C0, CH), jnp.int32),
            pltpu.VMEM((C0, CH), jnp.int32),
            pltpu.VMEM((CH, D), _f32),
            pltpu.VMEM_SHARED((NPAD, D), _f32),
        ],
    )


def _hop(ua, ub, src_p, dst_p):
    return _hop_kernel()(ua, ub, src_p, dst_p)


def _zk_body(x_ref, w_ref, degt_ref, u0_ref, u0b_ref, dinv_ref):
    dt = degt_ref[...]
    degsum = dt[:, 0:1] + dt[:, 1:2] + 1.0
    dinv = lax.rsqrt(degsum)
    z = jnp.dot(x_ref[...], w_ref[...], preferred_element_type=_f32)
    u0 = z * dinv
    u0_ref[...] = u0
    u0b_ref[...] = u0
    dinv_ref[...] = dinv


def _zk(x_p, w_conv, degt):
    return pl.pallas_call(
        _zk_body,
        grid=(NBLK,),
        in_specs=[
            pl.BlockSpec((128, D), lambda i: (i, 0)),
            pl.BlockSpec((D, D), lambda i: (0, 0)),
            pl.BlockSpec((128, NC), lambda i: (i, 0)),
        ],
        out_specs=[
            pl.BlockSpec((128, D), lambda i: (i, 0)),
            pl.BlockSpec((128, D), lambda i: (i, 0)),
            pl.BlockSpec((128, 1), lambda i: (i, 0)),
        ],
        out_shape=[
            jax.ShapeDtypeStruct((NPAD, D), _f32),
            jax.ShapeDtypeStruct((NPAD, D), _f32),
            jax.ShapeDtypeStruct((NPAD, 1), _f32),
        ],
    )(x_p, w_conv, degt)


def _mid_body(ap_ref, u0_ref, dinv_ref, u1_ref, u1b_ref):
    a = ap_ref[0] + ap_ref[1]
    dinv = dinv_ref[...]
    u1 = (a + u0_ref[...]) * (dinv * dinv)
    u1_ref[...] = u1
    u1b_ref[...] = u1


def _mid(a_partials, u0, dinv):
    return pl.pallas_call(
        _mid_body,
        grid=(NBLK,),
        in_specs=[
            pl.BlockSpec((NC, 128, D), lambda i: (0, i, 0)),
            pl.BlockSpec((128, D), lambda i: (i, 0)),
            pl.BlockSpec((128, 1), lambda i: (i, 0)),
        ],
        out_specs=[
            pl.BlockSpec((128, D), lambda i: (i, 0)),
            pl.BlockSpec((128, D), lambda i: (i, 0)),
        ],
        out_shape=[
            jax.ShapeDtypeStruct((NPAD, D), _f32),
            jax.ShapeDtypeStruct((NPAD, D), _f32),
        ],
    )(a_partials, u0, dinv)


def _fin_body(ap_ref, u1_ref, dinv_ref, bc_ref, wp_ref, bp_ref, out_ref,
              acc_ref):
    i = pl.program_id(0)

    @pl.when(i == 0)
    def _():
        acc_ref[...] = jnp.zeros_like(acc_ref)

    a = ap_ref[0] + ap_ref[1]
    h = (a + u1_ref[...]) * dinv_ref[...]
    h = jnp.maximum(h + bc_ref[...], 0.0)
    row = i * 128 + lax.broadcasted_iota(jnp.int32, (128, 1), 0)
    h = jnp.where(row < N_NODES, h, 0.0)
    acc_ref[...] += jnp.sum(h, axis=0, keepdims=True)

    @pl.when(i == pl.num_programs(0) - 1)
    def _():
        g = acc_ref[...] * (1.0 / N_NODES)
        out_ref[...] = (
            jnp.dot(g, wp_ref[...], preferred_element_type=_f32) + bp_ref[...]
        )


def _fin(a_partials, u1, dinv, b_conv2, wp_pad, bp_pad):
    return pl.pallas_call(
        _fin_body,
        grid=(NBLK,),
        in_specs=[
            pl.BlockSpec((NC, 128, D), lambda i: (0, i, 0)),
            pl.BlockSpec((128, D), lambda i: (i, 0)),
            pl.BlockSpec((128, 1), lambda i: (i, 0)),
            pl.BlockSpec((1, D), lambda i: (0, 0)),
            pl.BlockSpec((D, D), lambda i: (0, 0)),
            pl.BlockSpec((1, D), lambda i: (0, 0)),
        ],
        out_specs=pl.BlockSpec((1, D), lambda i: (0, 0)),
        out_shape=jax.ShapeDtypeStruct((1, D), _f32),
        scratch_shapes=[pltpu.VMEM((1, D), _f32)],
    )(a_partials, u1, dinv, b_conv2, wp_pad, bp_pad)


def kernel(nfeat, edge_index, W_conv, b_conv, W_pred, b_pred):
    src = edge_index[0]
    dst = edge_index[1]
    pad = jnp.full((EPAD - N_EDGES,), N_NODES, jnp.int32)
    src_p = jnp.concatenate([src, pad]).reshape(TOTP, CH)
    dst_p = jnp.concatenate([dst, pad]).reshape(TOTP, CH)
    dst_d = dst_p.reshape(NW, G_CHUNKS, CH)
    x_p = jnp.zeros((NPAD, D), _f32).at[:N_NODES].set(nfeat)
    b_conv2 = b_conv.reshape(1, D)
    wp_pad = jnp.zeros((D, D), _f32).at[:, :OUT].set(W_pred)
    bp_pad = jnp.zeros((1, D), _f32).at[0, :OUT].set(b_pred)

    deg_p = _deg(dst_d)
    degt = deg_p.T
    u0, u0b, dinv = _zk(x_p, W_conv, degt)
    a1 = _hop(u0, u0b, src_p, dst_p)
    u1, u1b = _mid(a1, u0, dinv)
    a2 = _hop(u1, u1b, src_p, dst_p)
    out128 = _fin(a2, u1, dinv, b_conv2, wp_pad, bp_pad)
    return out128[:, :OUT]


# speedup vs baseline: 1.4115x; 1.4115x over previous
import functools

import jax
import jax.numpy as jnp
from jax import lax
from jax.experimental import pallas as pl
from jax.experimental.pallas import tpu as pltpu
from jax.experimental.pallas import tpu_sc as plsc

N_NODES = 10000
N_EDGES = 320000
D = 128
OUT = 3

NC = 2
NS = 16
NW = NC * NS
CH = 128
G_CHUNKS = -(-N_EDGES // (NW * CH))
EPT = G_CHUNKS * CH
EPAD = EPT * NW
NPAD = 10240
RPT = NPAD // NS
RB = RPT // CH
NBLK = NPAD // 128

_f32 = jnp.float32


def _zero16():
    return jnp.zeros((16,), _f32)


def _one16():
    return jnp.ones((16,), _f32)


def _mesh():
    return plsc.VectorSubcoreMesh(
        core_axis_name="c", subcore_axis_name="s", num_cores=NC,
        num_subcores=NS,
    )


def _deg_body(dst_hbm, out_hbm, idx_v, ones_v, zeros_v, deg_sh):
    c = lax.axis_index("c")
    s = lax.axis_index("s")
    w = c * NS + s
    pltpu.sync_copy(dst_hbm.at[w], idx_v)
    for i in range(CH // 16):
        ones_v[pl.ds(i * 16, 16)] = _one16()
    for i in range(RPT // 16):
        zeros_v[pl.ds(i * 16, 16)] = _zero16()
    pltpu.sync_copy(zeros_v, deg_sh.at[pl.ds(s * RPT, RPT)])
    plsc.subcore_barrier()

    def chunk(g, carry):
        pltpu.sync_copy(ones_v, deg_sh.at[idx_v.at[g]], add=True)
        return carry

    lax.fori_loop(0, G_CHUNKS, chunk, 0)
    plsc.subcore_barrier()
    pltpu.sync_copy(deg_sh.at[pl.ds(s * RPT, RPT)],
                    out_hbm.at[c, pl.ds(s * RPT, RPT)])


@functools.cache
def _deg_kernel():
    return pl.kernel(
        _deg_body,
        out_type=jax.ShapeDtypeStruct((NC, NPAD), _f32),
        mesh=_mesh(),
        scratch_types=[
            pltpu.VMEM((G_CHUNKS, CH), jnp.int32),
            pltpu.VMEM((CH,), _f32),
            pltpu.VMEM((RPT,), _f32),
            pltpu.VMEM_SHARED((NPAD,), _f32),
        ],
    )


def _deg(dst_p):
    return _deg_kernel()(dst_p)


def _hop_body(u_hbm, src_hbm, dst_hbm, out_hbm, sidx_v, didx_v, rows_v,
              acc_sh):
    c = lax.axis_index("c")
    s = lax.axis_index("s")
    w = c * NS + s
    pltpu.sync_copy(src_hbm.at[w], sidx_v)
    pltpu.sync_copy(dst_hbm.at[w], didx_v)

    def zrow(i, carry):
        for k in range(D // 16):
            rows_v[i, pl.ds(k * 16, 16)] = _zero16()
        return carry

    lax.fori_loop(0, CH, zrow, 0)
    for r in range(RB):
        pltpu.sync_copy(rows_v, acc_sh.at[pl.ds(s * RPT + r * CH, CH)])
    plsc.subcore_barrier()

    def chunk(g, carry):
        pltpu.sync_copy(u_hbm.at[sidx_v.at[g]], rows_v)
        pltpu.sync_copy(rows_v, acc_sh.at[didx_v.at[g]], add=True)
        return carry

    lax.fori_loop(0, G_CHUNKS, chunk, 0)
    plsc.subcore_barrier()
    for r in range(RB):
        pltpu.sync_copy(acc_sh.at[pl.ds(s * RPT + r * CH, CH)],
                        out_hbm.at[c, pl.ds(s * RPT + r * CH, CH)])


@functools.cache
def _hop_kernel():
    return pl.kernel(
        _hop_body,
        out_type=jax.ShapeDtypeStruct((NC, NPAD, D), _f32),
        mesh=_mesh(),
        scratch_types=[
            pltpu.VMEM((G_CHUNKS, CH), jnp.int32),
            pltpu.VMEM((G_CHUNKS, CH), jnp.int32),
            pltpu.VMEM((CH, D), _f32),
            pltpu.VMEM_SHARED((NPAD, D), _f32),
        ],
    )


def _hop(u, src_p, dst_p):
    return _hop_kernel()(u, src_p, dst_p)


def _zk_body(x_ref, w_ref, degt_ref, u0_ref, dinv_ref):
    dt = degt_ref[...]
    degsum = dt[:, 0:1] + dt[:, 1:2] + 1.0
    dinv = lax.rsqrt(degsum)
    z = jnp.dot(x_ref[...], w_ref[...], preferred_element_type=_f32)
    u0_ref[...] = z * dinv
    dinv_ref[...] = dinv


def _zk(x_p, w_conv, degt):
    return pl.pallas_call(
        _zk_body,
        grid=(NBLK,),
        in_specs=[
            pl.BlockSpec((128, D), lambda i: (i, 0)),
            pl.BlockSpec((D, D), lambda i: (0, 0)),
            pl.BlockSpec((128, NC), lambda i: (i, 0)),
        ],
        out_specs=[
            pl.BlockSpec((128, D), lambda i: (i, 0)),
            pl.BlockSpec((128, 1), lambda i: (i, 0)),
        ],
        out_shape=[
            jax.ShapeDtypeStruct((NPAD, D), _f32),
            jax.ShapeDtypeStruct((NPAD, 1), _f32),
        ],
    )(x_p, w_conv, degt)


def _mid_body(ap_ref, u0_ref, dinv_ref, u1_ref):
    a = ap_ref[0] + ap_ref[1]
    dinv = dinv_ref[...]
    u1_ref[...] = (a + u0_ref[...]) * (dinv * dinv)


def _mid(a_partials, u0, dinv):
    return pl.pallas_call(
        _mid_body,
        grid=(NBLK,),
        in_specs=[
            pl.BlockSpec((NC, 128, D), lambda i: (0, i, 0)),
            pl.BlockSpec((128, D), lambda i: (i, 0)),
            pl.BlockSpec((128, 1), lambda i: (i, 0)),
        ],
        out_specs=pl.BlockSpec((128, D), lambda i: (i, 0)),
        out_shape=jax.ShapeDtypeStruct((NPAD, D), _f32),
    )(a_partials, u0, dinv)


def _fin_body(ap_ref, u1_ref, dinv_ref, bc_ref, wp_ref, bp_ref, out_ref,
              acc_ref):
    i = pl.program_id(0)

    @pl.when(i == 0)
    def _():
        acc_ref[...] = jnp.zeros_like(acc_ref)

    a = ap_ref[0] + ap_ref[1]
    h = (a + u1_ref[...]) * dinv_ref[...]
    h = jnp.maximum(h + bc_ref[...], 0.0)
    row = i * 128 + lax.broadcasted_iota(jnp.int32, (128, 1), 0)
    h = jnp.where(row < N_NODES, h, 0.0)
    acc_ref[...] += jnp.sum(h, axis=0, keepdims=True)

    @pl.when(i == pl.num_programs(0) - 1)
    def _():
        g = acc_ref[...] * (1.0 / N_NODES)
        out_ref[...] = (
            jnp.dot(g, wp_ref[...], preferred_element_type=_f32) + bp_ref[...]
        )


def _fin(a_partials, u1, dinv, b_conv2, wp_pad, bp_pad):
    return pl.pallas_call(
        _fin_body,
        grid=(NBLK,),
        in_specs=[
            pl.BlockSpec((NC, 128, D), lambda i: (0, i, 0)),
            pl.BlockSpec((128, D), lambda i: (i, 0)),
            pl.BlockSpec((128, 1), lambda i: (i, 0)),
            pl.BlockSpec((1, D), lambda i: (0, 0)),
            pl.BlockSpec((D, D), lambda i: (0, 0)),
            pl.BlockSpec((1, D), lambda i: (0, 0)),
        ],
        out_specs=pl.BlockSpec((1, D), lambda i: (0, 0)),
        out_shape=jax.ShapeDtypeStruct((1, D), _f32),
        scratch_shapes=[pltpu.VMEM((1, D), _f32)],
    )(a_partials, u1, dinv, b_conv2, wp_pad, bp_pad)


def kernel(nfeat, edge_index, W_conv, b_conv, W_pred, b_pred):
    src = edge_index[0]
    dst = edge_index[1]
    pad = jnp.full((EPAD - N_EDGES,), N_NODES, jnp.int32)
    src_p = jnp.concatenate([src, pad]).reshape(NW, G_CHUNKS, CH)
    dst_p = jnp.concatenate([dst, pad]).reshape(NW, G_CHUNKS, CH)
    x_p = jnp.zeros((NPAD, D), _f32).at[:N_NODES].set(nfeat)
    b_conv2 = b_conv.reshape(1, D)
    wp_pad = jnp.zeros((D, D), _f32).at[:, :OUT].set(W_pred)
    bp_pad = jnp.zeros((1, D), _f32).at[0, :OUT].set(b_pred)

    deg_p = _deg(dst_p)
    degt = deg_p.T
    u0, dinv = _zk(x_p, W_conv, degt)
    a1 = _hop(u0, src_p, dst_p)
    u1 = _mid(a1, u0, dinv)
    a2 = _hop(u1, src_p, dst_p)
    out128 = _fin(a2, u1, dinv, b_conv2, wp_pad, bp_pad)
    return out128[:, :OUT]


# speedup vs baseline: 1.5672x; 1.1102x over previous
import functools

import jax
import jax.numpy as jnp
from jax import lax
from jax.experimental import pallas as pl
from jax.experimental.pallas import tpu as pltpu
from jax.experimental.pallas import tpu_sc as plsc

N_NODES = 10000
N_EDGES = 320000
D = 128
OUT = 3

NC = 2
NS = 16
NW = NC * NS
CH = 128
G_CHUNKS = -(-N_EDGES // (NW * CH))
EPT = G_CHUNKS * CH
EPAD = EPT * NW
NPAD = 10240
RPT = NPAD // NS
RB = RPT // CH
RBLK = 512
NBLK = NPAD // RBLK

_f32 = jnp.float32


def _zero16():
    return jnp.zeros((16,), _f32)


def _one16():
    return jnp.ones((16,), _f32)


def _mesh():
    return plsc.VectorSubcoreMesh(
        core_axis_name="c", subcore_axis_name="s", num_cores=NC,
        num_subcores=NS,
    )


def _deg_body(dst_hbm, out_hbm, idx_v, ones_v, zeros_v, deg_sh):
    c = lax.axis_index("c")
    s = lax.axis_index("s")
    w = c * NS + s
    pltpu.sync_copy(dst_hbm.at[w], idx_v)
    for i in range(CH // 16):
        ones_v[pl.ds(i * 16, 16)] = _one16()
    for i in range(RPT // 16):
        zeros_v[pl.ds(i * 16, 16)] = _zero16()
    pltpu.sync_copy(zeros_v, deg_sh.at[pl.ds(s * RPT, RPT)])
    plsc.subcore_barrier()

    def chunk(g, carry):
        pltpu.sync_copy(ones_v, deg_sh.at[idx_v.at[g]], add=True)
        return carry

    lax.fori_loop(0, G_CHUNKS, chunk, 0)
    plsc.subcore_barrier()
    pltpu.sync_copy(deg_sh.at[pl.ds(s * RPT, RPT)],
                    out_hbm.at[c, pl.ds(s * RPT, RPT)])


@functools.cache
def _deg_kernel():
    return pl.kernel(
        _deg_body,
        out_type=jax.ShapeDtypeStruct((NC, NPAD), _f32),
        mesh=_mesh(),
        scratch_types=[
            pltpu.VMEM((G_CHUNKS, CH), jnp.int32),
            pltpu.VMEM((CH,), _f32),
            pltpu.VMEM((RPT,), _f32),
            pltpu.VMEM_SHARED((NPAD,), _f32),
        ],
    )


def _deg(dst_p):
    return _deg_kernel()(dst_p)


def _hop_body(u_hbm, src_hbm, dst_hbm, out_hbm, sidx_v, didx_v, rows_v,
              acc_sh):
    c = lax.axis_index("c")
    s = lax.axis_index("s")
    w = c * NS + s
    pltpu.sync_copy(src_hbm.at[w], sidx_v)
    pltpu.sync_copy(dst_hbm.at[w], didx_v)

    def zrow(i, carry):
        for k in range(D // 16):
            rows_v[i, pl.ds(k * 16, 16)] = _zero16()
        return carry

    lax.fori_loop(0, CH, zrow, 0)
    for r in range(RB):
        pltpu.sync_copy(rows_v, acc_sh.at[pl.ds(s * RPT + r * CH, CH)])
    plsc.subcore_barrier()

    def chunk(g, carry):
        pltpu.sync_copy(u_hbm.at[sidx_v.at[g]], rows_v)
        pltpu.sync_copy(rows_v, acc_sh.at[didx_v.at[g]], add=True)
        return carry

    lax.fori_loop(0, G_CHUNKS, chunk, 0)
    plsc.subcore_barrier()
    for r in range(RB):
        pltpu.sync_copy(acc_sh.at[pl.ds(s * RPT + r * CH, CH)],
                        out_hbm.at[c, pl.ds(s * RPT + r * CH, CH)])


@functools.cache
def _hop_kernel():
    return pl.kernel(
        _hop_body,
        out_type=jax.ShapeDtypeStruct((NC, NPAD, D), _f32),
        mesh=_mesh(),
        scratch_types=[
            pltpu.VMEM((G_CHUNKS, CH), jnp.int32),
            pltpu.VMEM((G_CHUNKS, CH), jnp.int32),
            pltpu.VMEM((CH, D), _f32),
            pltpu.VMEM_SHARED((NPAD, D), _f32),
        ],
    )


def _hop(u, src_p, dst_p):
    return _hop_kernel()(u, src_p, dst_p)


def _zk_body(x_ref, w_ref, degt_ref, u0_ref, dinv_ref):
    dt = degt_ref[...]
    degsum = dt[:, 0:1] + dt[:, 1:2] + 1.0
    dinv = lax.rsqrt(degsum)
    z = jnp.dot(x_ref[...], w_ref[...], preferred_element_type=_f32)
    u0_ref[...] = z * dinv
    dinv_ref[...] = dinv


def _zk(x_p, w_conv, degt):
    return pl.pallas_call(
        _zk_body,
        grid=(NBLK,),
        in_specs=[
            pl.BlockSpec((RBLK, D), lambda i: (i, 0)),
            pl.BlockSpec((D, D), lambda i: (0, 0)),
            pl.BlockSpec((RBLK, NC), lambda i: (i, 0)),
        ],
        out_specs=[
            pl.BlockSpec((RBLK, D), lambda i: (i, 0)),
            pl.BlockSpec((RBLK, 1), lambda i: (i, 0)),
        ],
        out_shape=[
            jax.ShapeDtypeStruct((NPAD, D), _f32),
            jax.ShapeDtypeStruct((NPAD, 1), _f32),
        ],
    )(x_p, w_conv, degt)


def _mid_body(ap_ref, u0_ref, dinv_ref, u1_ref):
    a = ap_ref[0] + ap_ref[1]
    dinv = dinv_ref[...]
    u1_ref[...] = (a + u0_ref[...]) * (dinv * dinv)


def _mid(a_partials, u0, dinv):
    return pl.pallas_call(
        _mid_body,
        grid=(NBLK,),
        in_specs=[
            pl.BlockSpec((NC, RBLK, D), lambda i: (0, i, 0)),
            pl.BlockSpec((RBLK, D), lambda i: (i, 0)),
            pl.BlockSpec((RBLK, 1), lambda i: (i, 0)),
        ],
        out_specs=pl.BlockSpec((RBLK, D), lambda i: (i, 0)),
        out_shape=jax.ShapeDtypeStruct((NPAD, D), _f32),
    )(a_partials, u0, dinv)


def _fin_body(ap_ref, u1_ref, dinv_ref, bc_ref, wp_ref, bp_ref, out_ref,
              acc_ref):
    i = pl.program_id(0)

    @pl.when(i == 0)
    def _():
        acc_ref[...] = jnp.zeros_like(acc_ref)

    a = ap_ref[0] + ap_ref[1]
    h = (a + u1_ref[...]) * dinv_ref[...]
    h = jnp.maximum(h + bc_ref[...], 0.0)
    row = i * RBLK + lax.broadcasted_iota(jnp.int32, (RBLK, 1), 0)
    h = jnp.where(row < N_NODES, h, 0.0)
    acc_ref[...] += jnp.sum(h, axis=0, keepdims=True)

    @pl.when(i == pl.num_programs(0) - 1)
    def _():
        g = acc_ref[...] * (1.0 / N_NODES)
        out_ref[...] = (
            jnp.dot(g, wp_ref[...], preferred_element_type=_f32) + bp_ref[...]
        )


def _fin(a_partials, u1, dinv, b_conv2, wp_pad, bp_pad):
    return pl.pallas_call(
        _fin_body,
        grid=(NBLK,),
        in_specs=[
            pl.BlockSpec((NC, RBLK, D), lambda i: (0, i, 0)),
            pl.BlockSpec((RBLK, D), lambda i: (i, 0)),
            pl.BlockSpec((RBLK, 1), lambda i: (i, 0)),
            pl.BlockSpec((1, D), lambda i: (0, 0)),
            pl.BlockSpec((D, D), lambda i: (0, 0)),
            pl.BlockSpec((1, D), lambda i: (0, 0)),
        ],
        out_specs=pl.BlockSpec((1, D), lambda i: (0, 0)),
        out_shape=jax.ShapeDtypeStruct((1, D), _f32),
        scratch_shapes=[pltpu.VMEM((1, D), _f32)],
    )(a_partials, u1, dinv, b_conv2, wp_pad, bp_pad)


def kernel(nfeat, edge_index, W_conv, b_conv, W_pred, b_pred):
    src = edge_index[0]
    dst = edge_index[1]
    pad = jnp.full((EPAD - N_EDGES,), N_NODES, jnp.int32)
    src_p = jnp.concatenate([src, pad]).reshape(NW, G_CHUNKS, CH)
    dst_p = jnp.concatenate([dst, pad]).reshape(NW, G_CHUNKS, CH)
    x_p = jnp.zeros((NPAD, D), _f32).at[:N_NODES].set(nfeat)
    b_conv2 = b_conv.reshape(1, D)
    wp_pad = jnp.zeros((D, D), _f32).at[:, :OUT].set(W_pred)
    bp_pad = jnp.zeros((1, D), _f32).at[0, :OUT].set(b_pred)

    deg_p = _deg(dst_p)
    degt = deg_p.T
    u0, dinv = _zk(x_p, W_conv, degt)
    a1 = _hop(u0, src_p, dst_p)
    u1 = _mid(a1, u0, dinv)
    a2 = _hop(u1, src_p, dst_p)
    out128 = _fin(a2, u1, dinv, b_conv2, wp_pad, bp_pad)
    return out128[:, :OUT]


# speedup vs baseline: 1.6095x; 1.0270x over previous
import functools

import jax
import jax.numpy as jnp
from jax import lax
from jax.experimental import pallas as pl
from jax.experimental.pallas import tpu as pltpu
from jax.experimental.pallas import tpu_sc as plsc

N_NODES = 10000
N_EDGES = 320000
D = 128
OUT = 3

NC = 2
NS = 16
NW = NC * NS
CH = 128
G_CHUNKS = -(-N_EDGES // (NW * CH))
EPT = G_CHUNKS * CH
EPAD = EPT * NW
NPAD = 10240
RPT = NPAD // NS
RB = RPT // CH
RBLK = 2048
NBLK = NPAD // RBLK

_f32 = jnp.float32


def _zero16():
    return jnp.zeros((16,), _f32)


def _one16():
    return jnp.ones((16,), _f32)


def _mesh():
    return plsc.VectorSubcoreMesh(
        core_axis_name="c", subcore_axis_name="s", num_cores=NC,
        num_subcores=NS,
    )


def _deg_body(dst_hbm, out_hbm, idx_v, ones_v, zeros_v, deg_sh):
    c = lax.axis_index("c")
    s = lax.axis_index("s")
    w = c * NS + s
    pltpu.sync_copy(dst_hbm.at[w], idx_v)
    for i in range(CH // 16):
        ones_v[pl.ds(i * 16, 16)] = _one16()
    for i in range(RPT // 16):
        zeros_v[pl.ds(i * 16, 16)] = _zero16()
    pltpu.sync_copy(zeros_v, deg_sh.at[pl.ds(s * RPT, RPT)])
    plsc.subcore_barrier()

    def chunk(g, carry):
        pltpu.sync_copy(ones_v, deg_sh.at[idx_v.at[g]], add=True)
        return carry

    lax.fori_loop(0, G_CHUNKS, chunk, 0)
    plsc.subcore_barrier()
    pltpu.sync_copy(deg_sh.at[pl.ds(s * RPT, RPT)],
                    out_hbm.at[c, pl.ds(s * RPT, RPT)])


@functools.cache
def _deg_kernel():
    return pl.kernel(
        _deg_body,
        out_type=jax.ShapeDtypeStruct((NC, NPAD), _f32),
        mesh=_mesh(),
        scratch_types=[
            pltpu.VMEM((G_CHUNKS, CH), jnp.int32),
            pltpu.VMEM((CH,), _f32),
            pltpu.VMEM((RPT,), _f32),
            pltpu.VMEM_SHARED((NPAD,), _f32),
        ],
    )


def _deg(dst_p):
    return _deg_kernel()(dst_p)


def _hop_body(u_hbm, src_hbm, dst_hbm, out_hbm, sidx_v, didx_v, rows_v,
              acc_sh):
    c = lax.axis_index("c")
    s = lax.axis_index("s")
    w = c * NS + s
    pltpu.sync_copy(src_hbm.at[w], sidx_v)
    pltpu.sync_copy(dst_hbm.at[w], didx_v)

    def zrow(i, carry):
        for k in range(D // 16):
            rows_v[i, pl.ds(k * 16, 16)] = _zero16()
        return carry

    lax.fori_loop(0, CH, zrow, 0)
    for r in range(RB):
        pltpu.sync_copy(rows_v, acc_sh.at[pl.ds(s * RPT + r * CH, CH)])
    plsc.subcore_barrier()

    def chunk(g, carry):
        pltpu.sync_copy(u_hbm.at[sidx_v.at[g]], rows_v)
        pltpu.sync_copy(rows_v, acc_sh.at[didx_v.at[g]], add=True)
        return carry

    lax.fori_loop(0, G_CHUNKS, chunk, 0)
    plsc.subcore_barrier()
    for r in range(RB):
        pltpu.sync_copy(acc_sh.at[pl.ds(s * RPT + r * CH, CH)],
                        out_hbm.at[c, pl.ds(s * RPT + r * CH, CH)])


@functools.cache
def _hop_kernel():
    return pl.kernel(
        _hop_body,
        out_type=jax.ShapeDtypeStruct((NC, NPAD, D), _f32),
        mesh=_mesh(),
        scratch_types=[
            pltpu.VMEM((G_CHUNKS, CH), jnp.int32),
            pltpu.VMEM((G_CHUNKS, CH), jnp.int32),
            pltpu.VMEM((CH, D), _f32),
            pltpu.VMEM_SHARED((NPAD, D), _f32),
        ],
    )


def _hop(u, src_p, dst_p):
    return _hop_kernel()(u, src_p, dst_p)


def _zk_body(x_ref, w_ref, degt_ref, u0_ref, dinv_ref):
    dt = degt_ref[...]
    degsum = dt[:, 0:1] + dt[:, 1:2] + 1.0
    dinv = lax.rsqrt(degsum)
    z = jnp.dot(x_ref[...], w_ref[...], preferred_element_type=_f32)
    u0_ref[...] = z * dinv
    dinv_ref[...] = dinv


def _zk(x_p, w_conv, degt):
    return pl.pallas_call(
        _zk_body,
        grid=(NBLK,),
        in_specs=[
            pl.BlockSpec((RBLK, D), lambda i: (i, 0)),
            pl.BlockSpec((D, D), lambda i: (0, 0)),
            pl.BlockSpec((RBLK, NC), lambda i: (i, 0)),
        ],
        out_specs=[
            pl.BlockSpec((RBLK, D), lambda i: (i, 0)),
            pl.BlockSpec((RBLK, 1), lambda i: (i, 0)),
        ],
        out_shape=[
            jax.ShapeDtypeStruct((NPAD, D), _f32),
            jax.ShapeDtypeStruct((NPAD, 1), _f32),
        ],
    )(x_p, w_conv, degt)


def _mid_body(ap_ref, u0_ref, dinv_ref, u1_ref):
    a = ap_ref[0] + ap_ref[1]
    dinv = dinv_ref[...]
    u1_ref[...] = (a + u0_ref[...]) * (dinv * dinv)


def _mid(a_partials, u0, dinv):
    return pl.pallas_call(
        _mid_body,
        grid=(NBLK,),
        in_specs=[
            pl.BlockSpec((NC, RBLK, D), lambda i: (0, i, 0)),
            pl.BlockSpec((RBLK, D), lambda i: (i, 0)),
            pl.BlockSpec((RBLK, 1), lambda i: (i, 0)),
        ],
        out_specs=pl.BlockSpec((RBLK, D), lambda i: (i, 0)),
        out_shape=jax.ShapeDtypeStruct((NPAD, D), _f32),
    )(a_partials, u0, dinv)


def _fin_body(ap_ref, u1_ref, dinv_ref, bc_ref, wp_ref, bp_ref, out_ref,
              acc_ref):
    i = pl.program_id(0)

    @pl.when(i == 0)
    def _():
        acc_ref[...] = jnp.zeros_like(acc_ref)

    a = ap_ref[0] + ap_ref[1]
    h = (a + u1_ref[...]) * dinv_ref[...]
    h = jnp.maximum(h + bc_ref[...], 0.0)
    row = i * RBLK + lax.broadcasted_iota(jnp.int32, (RBLK, 1), 0)
    h = jnp.where(row < N_NODES, h, 0.0)
    acc_ref[...] += jnp.sum(h, axis=0, keepdims=True)

    @pl.when(i == pl.num_programs(0) - 1)
    def _():
        g = acc_ref[...] * (1.0 / N_NODES)
        out_ref[...] = (
            jnp.dot(g, wp_ref[...], preferred_element_type=_f32) + bp_ref[...]
        )


def _fin(a_partials, u1, dinv, b_conv2, wp_pad, bp_pad):
    return pl.pallas_call(
        _fin_body,
        grid=(NBLK,),
        in_specs=[
            pl.BlockSpec((NC, RBLK, D), lambda i: (0, i, 0)),
            pl.BlockSpec((RBLK, D), lambda i: (i, 0)),
            pl.BlockSpec((RBLK, 1), lambda i: (i, 0)),
            pl.BlockSpec((1, D), lambda i: (0, 0)),
            pl.BlockSpec((D, D), lambda i: (0, 0)),
            pl.BlockSpec((1, D), lambda i: (0, 0)),
        ],
        out_specs=pl.BlockSpec((1, D), lambda i: (0, 0)),
        out_shape=jax.ShapeDtypeStruct((1, D), _f32),
        scratch_shapes=[pltpu.VMEM((1, D), _f32)],
    )(a_partials, u1, dinv, b_conv2, wp_pad, bp_pad)


def kernel(nfeat, edge_index, W_conv, b_conv, W_pred, b_pred):
    src = edge_index[0]
    dst = edge_index[1]
    pad = jnp.full((EPAD - N_EDGES,), N_NODES, jnp.int32)
    src_p = jnp.concatenate([src, pad]).reshape(NW, G_CHUNKS, CH)
    dst_p = jnp.concatenate([dst, pad]).reshape(NW, G_CHUNKS, CH)
    x_p = jnp.zeros((NPAD, D), _f32).at[:N_NODES].set(nfeat)
    b_conv2 = b_conv.reshape(1, D)
    wp_pad = jnp.zeros((D, D), _f32).at[:, :OUT].set(W_pred)
    bp_pad = jnp.zeros((1, D), _f32).at[0, :OUT].set(b_pred)

    deg_p = _deg(dst_p)
    degt = deg_p.T
    u0, dinv = _zk(x_p, W_conv, degt)
    a1 = _hop(u0, src_p, dst_p)
    u1 = _mid(a1, u0, dinv)
    a2 = _hop(u1, src_p, dst_p)
    out128 = _fin(a2, u1, dinv, b_conv2, wp_pad, bp_pad)
    return out128[:, :OUT]
